# baseline (speedup 1.0000x reference)
import functools

import jax
import jax.numpy as jnp
from jax import lax
from jax.experimental import pallas as pl
from jax.experimental.pallas import tpu as pltpu
from jax.experimental.pallas import tpu_sc as plsc

_B = 4096
_L = 200
_D = 64
_V = 1000
_SPLITS = ((0, 104), (104, 96))
_info = plsc.get_sparse_core_info()
_NC = _info.num_cores
_NS = _info.num_subcores
_NW = _NC * _NS
_RPW = _B // _NW
_K = _RPW
_NBUF = 4
_NGRP = _K // _NBUF

_mesh = plsc.VectorSubcoreMesh(core_axis_name="c", subcore_axis_name="s")


@functools.partial(
    pl.kernel,
    mesh=_mesh,
    out_type=jax.ShapeDtypeStruct((_B, _L, _D), jnp.float32),
    scratch_types=[
        pltpu.VMEM((_RPW, _L), jnp.int32),
        pltpu.VMEM((_NBUF, _L, _D), jnp.float32),
        pltpu.VMEM_SHARED((_V, _D), jnp.float32),
        pltpu.SemaphoreType.DMA((_NBUF,)),
        pltpu.SemaphoreType.DMA((_NBUF,)),
    ],
    compiler_params=pltpu.CompilerParams(use_tc_tiling_on_sc=False),
)
def _sc_gather(idx_hbm, table_hbm, out_hbm, idx_v, rows_v, tab_s, gsem, osem):
    sid = lax.axis_index("s")
    wid = sid * _NC + lax.axis_index("c")
    row0 = wid * _RPW

    @pl.when(sid == 0)
    def _():
        pltpu.sync_copy(table_hbm, tab_s)

    pltpu.sync_copy(idx_hbm.at[pl.ds(row0, _RPW)], idx_v)
    plsc.subcore_barrier()

    def start_gather(j, b):
        for h, g in _SPLITS:
            pltpu.async_copy(
                tab_s.at[idx_v.at[j, pl.ds(h, g)]],
                rows_v.at[b, pl.ds(h, g)], gsem.at[b])

    def wait_gather(j, b):
        for h, g in _SPLITS:
            pltpu.make_async_copy(
                tab_s.at[idx_v.at[j, pl.ds(h, g)]],
                rows_v.at[b, pl.ds(h, g)], gsem.at[b]).wait()

    def start_scatter(j, b):
        pltpu.async_copy(rows_v.at[b], out_hbm.at[row0 + j], osem.at[b])

    def wait_scatter(j, b):
        pltpu.make_async_copy(
            rows_v.at[b], out_hbm.at[row0 + j], osem.at[b]).wait()

    for b in range(_NBUF):
        start_gather(b, b)

    def group(g, carry):
        for b in range(_NBUF):
            j = g * _NBUF + b
            wait_gather(j, b)
            start_scatter(j, b)
        for b in range(_NBUF):
            j = g * _NBUF + b
            wait_scatter(j, b)
            start_gather(j + _NBUF, b)
        return carry

    lax.fori_loop(0, _NGRP - 1, group, 0)

    for b in range(_NBUF):
        j = (_NGRP - 1) * _NBUF + b
        wait_gather(j, b)
        start_scatter(j, b)
    for b in range(_NBUF):
        j = (_NGRP - 1) * _NBUF + b
        wait_scatter(j, b)


def kernel(visit_segments, embedding_table):
    return _sc_gather(visit_segments, embedding_table)
